# baseline (speedup 1.0000x reference)
import functools

import jax
import jax.numpy as jnp
from jax import lax
from jax.experimental import pallas as pl
from jax.experimental.pallas import tpu as pltpu
from jax.experimental.pallas import tpu_sc as plsc

_info = plsc.get_sparse_core_info()
_NC, _NS, _L = _info.num_cores, _info.num_subcores, _info.num_lanes
_NW = _NC * _NS
_BPW = 3200
_NB_PAD = _NW * _BPW
_NVEC = _BPW // _L


def _sc_energy(cflat, idx0, idx1, b0p, kbp):
    mesh = plsc.VectorSubcoreMesh(core_axis_name="c", subcore_axis_name="s")

    @functools.partial(
        pl.kernel,
        out_type=jax.ShapeDtypeStruct((_NW, _L), jnp.float32),
        mesh=mesh,
        compiler_params=pltpu.CompilerParams(
            needs_layout_passes=False, use_tc_tiling_on_sc=False),
        scratch_types=[
            pltpu.VMEM((_BPW,), jnp.int32),
            pltpu.VMEM((_BPW,), jnp.int32),
            pltpu.VMEM((_BPW,), jnp.int32),
            pltpu.VMEM((_BPW,), jnp.int32),
            pltpu.VMEM((_BPW,), jnp.int32),
            pltpu.VMEM((_BPW,), jnp.int32),
            pltpu.VMEM((_BPW,), jnp.int32),
            pltpu.VMEM((_BPW,), jnp.int32),
            pltpu.VMEM((_BPW,), jnp.float32),
            pltpu.VMEM((_BPW,), jnp.float32),
            pltpu.VMEM((_BPW,), jnp.float32),
            pltpu.VMEM((_BPW,), jnp.float32),
            pltpu.VMEM((_BPW,), jnp.float32),
            pltpu.VMEM((_BPW,), jnp.float32),
            pltpu.VMEM((_BPW,), jnp.float32),
            pltpu.VMEM((_BPW,), jnp.float32),
            pltpu.VMEM((_L,), jnp.float32),
            pltpu.SemaphoreType.DMA,
            pltpu.SemaphoreType.DMA,
        ],
    )
    def k(cflat_h, i0_h, i1_h, b0_h, kb_h, out_h,
          i0_v, i1_v, ix0, iy0, iz0, ix1, iy1, iz1,
          x0, y0, z0, x1, y1, z1, b0_v, kb_v, acc_v, sem_a, sem_b):
        wid = lax.axis_index("s") * _NC + lax.axis_index("c")
        base = wid * _BPW
        half = _BPW // 2
        pltpu.sync_copy(i0_h.at[pl.ds(base, _BPW)], i0_v)
        pltpu.sync_copy(i1_h.at[pl.ds(base, _BPW)], i1_v)

        def build(t, carry):
            off = t * _L
            v0 = i0_v[pl.ds(off, _L)] * 3
            ix0[pl.ds(off, _L)] = v0
            iy0[pl.ds(off, _L)] = v0 + 1
            iz0[pl.ds(off, _L)] = v0 + 2
            v1 = i1_v[pl.ds(off, _L)] * 3
            ix1[pl.ds(off, _L)] = v1
            iy1[pl.ds(off, _L)] = v1 + 1
            iz1[pl.ds(off, _L)] = v1 + 2
            return carry

        def fire(sem, lo, n):
            cps = []
            for iv, dv in ((ix0, x0), (iy0, y0), (iz0, z0),
                           (ix1, x1), (iy1, y1), (iz1, z1)):
                cps.append(pltpu.async_copy(
                    cflat_h.at[iv.at[pl.ds(lo, n)]],
                    dv.at[pl.ds(lo, n)], sem))
            return cps

        lax.fori_loop(0, _NVEC // 2, build, 0)
        cps_a = fire(sem_a, 0, half)
        lax.fori_loop(_NVEC // 2, _NVEC, build, 0)
        cps_b = fire(sem_b, half, half)
        pltpu.sync_copy(b0_h.at[pl.ds(base, _BPW)], b0_v)
        pltpu.sync_copy(kb_h.at[pl.ds(base, _BPW)], kb_v)

        def body(t, acc):
            off = t * _L
            dx = x0[pl.ds(off, _L)] - x1[pl.ds(off, _L)]
            dy = y0[pl.ds(off, _L)] - y1[pl.ds(off, _L)]
            dz = z0[pl.ds(off, _L)] - z1[pl.ds(off, _L)]
            s = dx * dx + dy * dy + dz * dz
            bits = lax.bitcast_convert_type(s, jnp.int32)
            y = lax.bitcast_convert_type(
                jnp.int32(0x5F3759DF) - (bits >> 1), jnp.float32)
            hs = 0.5 * s
            y = y * (1.5 - hs * y * y)
            y = y * (1.5 - hs * y * y)
            y = y * (1.5 - hs * y * y)
            r = s * y
            kb = kb_v[pl.ds(off, _L)]
            d = r - b0_v[pl.ds(off, _L)]
            return acc + (0.5 * kb) * (d * d)

        for cp in cps_a:
            cp.wait()
        acc = lax.fori_loop(0, _NVEC // 2, body, jnp.zeros((_L,), jnp.float32))
        for cp in cps_b:
            cp.wait()
        acc = lax.fori_loop(_NVEC // 2, _NVEC, body, acc)
        acc_v[...] = acc
        pltpu.sync_copy(acc_v, out_h.at[wid])

    return k(cflat, idx0, idx1, b0p, kbp)


def kernel(coords, box, bonds, b0, k_bond):
    del box
    cflat = jnp.reshape(coords, (-1,))
    nb = b0.shape[0]
    pad = _NB_PAD - nb
    idx0 = jnp.pad(bonds[:, 0], (0, pad))
    idx1 = jnp.pad(bonds[:, 1], (0, pad))
    b0p = jnp.pad(b0, (0, pad))
    kbp = jnp.pad(k_bond, (0, pad))
    partials = _sc_energy(cflat, idx0, idx1, b0p, kbp)
    return jnp.sum(partials)


# speedup vs baseline: 1.4349x; 1.4349x over previous
import functools

import jax
import jax.numpy as jnp
from jax import lax
from jax.experimental import pallas as pl
from jax.experimental.pallas import tpu as pltpu
from jax.experimental.pallas import tpu_sc as plsc

_info = plsc.get_sparse_core_info()
_NC, _NS, _L = _info.num_cores, _info.num_subcores, _info.num_lanes
_NW = _NC * _NS
_BPW = 3200
_NB_PAD = _NW * _BPW
_NVEC = _BPW // _L
_APT = 3136
_EPT = _APT * 3
_NA8 = _APT * _NW
_NEL = _EPT * _NW

_PARAMS = pltpu.CompilerParams(
    needs_layout_passes=False, use_tc_tiling_on_sc=False)


def _sc_interleave(cpad):
    mesh = plsc.VectorSubcoreMesh(core_axis_name="c", subcore_axis_name="s")

    @functools.partial(
        pl.kernel,
        out_type=jax.ShapeDtypeStruct((_NA8, 8), jnp.float32),
        mesh=mesh,
        compiler_params=_PARAMS,
        scratch_types=[
            pltpu.VMEM((_EPT,), jnp.float32),
            pltpu.VMEM((_APT, 8), jnp.float32),
        ],
    )
    def k(cflat_h, out_h, e_v, rows_v):
        wid = lax.axis_index("s") * _NC + lax.axis_index("c")
        pltpu.sync_copy(cflat_h.at[pl.ds(wid * _EPT, _EPT)], e_v)
        iota = lax.iota(jnp.int32, _L)

        def body(t, carry):
            off = t * _L
            e = off + iota
            v = e_v[pl.ds(off, _L)]
            a = (e * 21846) >> 16
            c = e - (a * 2 + a)
            plsc.store_scatter(rows_v, [a, c], v)
            return carry

        lax.fori_loop(0, _EPT // _L, body, 0)
        pltpu.sync_copy(rows_v, out_h.at[pl.ds(wid * _APT, _APT)])

    return k(cpad)


def _sc_energy(coords8, idx0, idx1, b0p, kbp):
    mesh = plsc.VectorSubcoreMesh(core_axis_name="c", subcore_axis_name="s")

    @functools.partial(
        pl.kernel,
        out_type=jax.ShapeDtypeStruct((_NW, _L), jnp.float32),
        mesh=mesh,
        compiler_params=_PARAMS,
        scratch_types=[
            pltpu.VMEM((_BPW,), jnp.int32),
            pltpu.VMEM((_BPW,), jnp.int32),
            pltpu.VMEM((_BPW, 8), jnp.float32),
            pltpu.VMEM((_BPW, 8), jnp.float32),
            pltpu.VMEM((_BPW,), jnp.float32),
            pltpu.VMEM((_BPW,), jnp.float32),
            pltpu.VMEM((_L,), jnp.float32),
            pltpu.SemaphoreType.DMA,
            pltpu.SemaphoreType.DMA,
        ],
    )
    def k(coords_h, i0_h, i1_h, b0_h, kb_h, out_h,
          i0_v, i1_v, ri_v, rj_v, b0_v, kb_v, acc_v, sem_i, sem_j):
        wid = lax.axis_index("s") * _NC + lax.axis_index("c")
        base = wid * _BPW
        half = _BPW // 2
        pltpu.sync_copy(i0_h.at[pl.ds(base, _BPW)], i0_v)
        pltpu.sync_copy(i1_h.at[pl.ds(base, _BPW)], i1_v)
        cp_ai = pltpu.async_copy(coords_h.at[i0_v.at[pl.ds(0, half)]],
                                 ri_v.at[pl.ds(0, half)], sem_i)
        cp_aj = pltpu.async_copy(coords_h.at[i1_v.at[pl.ds(0, half)]],
                                 rj_v.at[pl.ds(0, half)], sem_i)
        cp_bi = pltpu.async_copy(coords_h.at[i0_v.at[pl.ds(half, half)]],
                                 ri_v.at[pl.ds(half, half)], sem_j)
        cp_bj = pltpu.async_copy(coords_h.at[i1_v.at[pl.ds(half, half)]],
                                 rj_v.at[pl.ds(half, half)], sem_j)
        pltpu.sync_copy(b0_h.at[pl.ds(base, _BPW)], b0_v)
        pltpu.sync_copy(kb_h.at[pl.ds(base, _BPW)], kb_v)

        iota = lax.iota(jnp.int32, _L)
        c0 = jnp.zeros((_L,), jnp.int32)
        c1 = c0 + 1
        c2 = c0 + 2

        def body(t, acc):
            b = t * _L + iota
            xi = plsc.load_gather(ri_v, [b, c0])
            yi = plsc.load_gather(ri_v, [b, c1])
            zi = plsc.load_gather(ri_v, [b, c2])
            xj = plsc.load_gather(rj_v, [b, c0])
            yj = plsc.load_gather(rj_v, [b, c1])
            zj = plsc.load_gather(rj_v, [b, c2])
            dx = xi - xj
            dy = yi - yj
            dz = zi - zj
            s = dx * dx + dy * dy + dz * dz
            bits = lax.bitcast_convert_type(s, jnp.int32)
            y = lax.bitcast_convert_type(
                jnp.int32(0x5F3759DF) - (bits >> 1), jnp.float32)
            hs = 0.5 * s
            y = y * (1.5 - hs * y * y)
            y = y * (1.5 - hs * y * y)
            y = y * (1.5 - hs * y * y)
            r = s * y
            off = t * _L
            kb = kb_v[pl.ds(off, _L)]
            d = r - b0_v[pl.ds(off, _L)]
            return acc + (0.5 * kb) * (d * d)

        cp_ai.wait()
        cp_aj.wait()
        acc = lax.fori_loop(0, _NVEC // 2, body, jnp.zeros((_L,), jnp.float32))
        cp_bi.wait()
        cp_bj.wait()
        acc = lax.fori_loop(_NVEC // 2, _NVEC, body, acc)
        acc_v[...] = acc
        pltpu.sync_copy(acc_v, out_h.at[wid])

    return k(coords8, idx0, idx1, b0p, kbp)


def kernel(coords, box, bonds, b0, k_bond):
    del box
    cpad = jnp.pad(jnp.reshape(coords, (-1,)), (0, _NEL - coords.size))
    coords8 = _sc_interleave(cpad)
    nb = b0.shape[0]
    pad = _NB_PAD - nb
    idx0 = jnp.pad(bonds[:, 0], (0, pad))
    idx1 = jnp.pad(bonds[:, 1], (0, pad))
    b0p = jnp.pad(b0, (0, pad))
    kbp = jnp.pad(k_bond, (0, pad))
    partials = _sc_energy(coords8, idx0, idx1, b0p, kbp)
    return jnp.sum(partials)
